# baseline (speedup 1.0000x reference)
import jax
import jax.numpy as jnp
from jax.experimental import pallas as pl
from jax.experimental.pallas import tpu as pltpu

B = 8192
D = 256
TEMP = 0.05
EPS = 1e-5
NORM_EPS = 1e-12
LOG2E = 1.4426950408889634
SC = LOG2E / TEMP
SQ = SC ** 0.5

BM = 1024
BN = 1024
NI = B // BM
NJ = B // BN
LN = 128


N_POLY = 2


def _exp2_poly(s):
    t = s + jnp.float32(12582912.0)
    n_f = t - jnp.float32(12582912.0)
    f = s - n_f
    bits = pltpu.bitcast(t, jnp.int32)
    e2n = pltpu.bitcast((bits << 23) + jnp.int32(127 << 23), jnp.float32)
    p = jnp.float32(0.0096181291)
    p = p * f + jnp.float32(0.0555041087)
    p = p * f + jnp.float32(0.2402265069)
    p = p * f + jnp.float32(0.6931471806)
    p = p * f + jnp.float32(1.0)
    return e2n * p


def _kprep_kernel(k_ref, kn8_ref):
    k = k_ref[...]
    ks = jnp.sum(k * k, axis=1, keepdims=True)
    kn = k * (1.0 / jnp.maximum(jnp.sqrt(ks), NORM_EPS))
    kn8_ref[...] = (kn * SQ).astype(jnp.float8_e4m3fn)


def _main_kernel(q_ref, kn8_ref, kb8_ref, o_ref, colp_ref, rl_ref, dlane_ref):
    i = pl.program_id(0)
    q = q_ref[...]
    qs = jnp.sum(q * q, axis=1, keepdims=True)
    qn = q * (1.0 / jnp.maximum(jnp.sqrt(qs), NORM_EPS))
    qb8 = (qn * SQ).astype(jnp.float8_e4m3fn)
    qk = qn * kb8_ref[...].astype(jnp.float32)
    dp = qk[:, :LN] + qk[:, LN:]
    dp_bf = dp.astype(jnp.bfloat16)

    rs = None
    cols = []
    for c in range(NJ):
        kb = kn8_ref[c * BN:(c + 1) * BN, :]
        s = jax.lax.dot_general(
            qb8, kb,
            (((1,), (1,)), ((), ())),
            preferred_element_type=jnp.float32,
        )
        if c < NJ - N_POLY:
            e = jnp.exp2(s)
        else:
            e = _exp2_poly(s)
        acc = e[:, 0:LN]
        for cc in range(1, BN // LN):
            acc = acc + e[:, cc * LN:(cc + 1) * LN]
        rs = acc if rs is None else rs + acc
        cols.append(jnp.sum(e, axis=0)[None, :])

    colp_ref[pl.ds(i, 1)] = jnp.concatenate(cols, axis=1)[None]

    ones = jnp.ones((LN, LN), jnp.bfloat16)
    den = jax.lax.dot_general(
        rs.astype(jnp.bfloat16), ones,
        (((1,), (0,)), ((), ())),
        preferred_element_type=jnp.float32,
    )
    drep = jax.lax.dot_general(
        dp_bf, ones,
        (((1,), (0,)), ((), ())),
        preferred_element_type=jnp.float32,
    )
    dexp = jnp.exp2(drep * SQ)
    lq = -jnp.log(dexp / den + EPS)
    rl_ref[pl.ds(i, 1)] = jnp.sum(lq, axis=0)[None, None, :]

    ones_row = jnp.ones((1, LN), jnp.bfloat16)
    dlane_ref[pl.ds(i, 1)] = jax.lax.dot_general(
        ones_row, dp_bf,
        (((1,), (1,)), ((), ())),
        preferred_element_type=jnp.float32,
    )[None]

    @pl.when(i == NI - 1)
    def _():
        den_kq = colp_ref[0, 0, :][None, :]
        for r in range(1, NI):
            den_kq = den_kq + colp_ref[r, 0, :][None, :]
        lk_sum = jnp.float32(0.0)
        for r in range(NI):
            dex = jnp.exp2(dlane_ref[r, 0, :][None, :] * SQ)
            seg = den_kq[:, r * BM:(r + 1) * BM]
            lk_sum = lk_sum + jnp.sum(-jnp.log(dex / seg + EPS))
        rl_sum = jnp.float32(0.0)
        for r in range(NI):
            rl_sum = rl_sum + jnp.sum(rl_ref[r, 0, :])
        o_ref[...] = jnp.reshape(
            (rl_sum * (1.0 / LN) + lk_sum) * (1.0 / B), (1, 1))


def kernel(q, k):
    kn8 = pl.pallas_call(
        _kprep_kernel,
        grid=(NI,),
        in_specs=[pl.BlockSpec((BM, D), lambda i: (i, 0))],
        out_specs=pl.BlockSpec((BM, D), lambda i: (i, 0)),
        out_shape=jax.ShapeDtypeStruct((B, D), jnp.float8_e4m3fn),
        compiler_params=pltpu.CompilerParams(
            dimension_semantics=("arbitrary",),
        ),
    )(k)

    loss = pl.pallas_call(
        _main_kernel,
        grid=(NI,),
        in_specs=[
            pl.BlockSpec((BM, D), lambda i: (i, 0)),
            pl.BlockSpec((B, D), lambda i: (0, 0)),
            pl.BlockSpec((BM, D), lambda i: (i, 0)),
        ],
        out_specs=pl.BlockSpec((1, 1), lambda i: (0, 0)),
        out_shape=jax.ShapeDtypeStruct((1, 1), jnp.float32),
        scratch_shapes=[
            pltpu.VMEM((NI, 1, B), jnp.float32),
            pltpu.VMEM((NI, 1, LN), jnp.float32),
            pltpu.VMEM((NI, 1, BM), jnp.float32),
        ],
        compiler_params=pltpu.CompilerParams(
            dimension_semantics=("arbitrary",),
            vmem_limit_bytes=40 * 1024 * 1024,
        ),
    )(q, kn8, kn8)

    return jnp.reshape(loss, ())


# speedup vs baseline: 1.5327x; 1.5327x over previous
import jax
import jax.numpy as jnp
from jax.experimental import pallas as pl
from jax.experimental.pallas import tpu as pltpu

B = 8192
D = 256
TEMP = 0.05
EPS = 1e-5
NORM_EPS = 1e-12
LOG2E = 1.4426950408889634
SC = LOG2E / TEMP
SQ = SC ** 0.5

BM = 1024
BN = 1024
NI = B // BM
NJ = B // BN
LN = 128


def _kprep_kernel(k_ref, kn8_ref):
    k = k_ref[...]
    ks = jnp.sum(k * k, axis=1, keepdims=True)
    kn = k * (1.0 / jnp.maximum(jnp.sqrt(ks), NORM_EPS))
    kn8_ref[...] = (kn * SQ).astype(jnp.float8_e4m3fn)


def _main_kernel(q_ref, kn8_ref, kb8_ref, o_ref, colp_ref, rl_ref, dlane_ref):
    i = pl.program_id(0)
    q = q_ref[...]
    qs = jnp.sum(q * q, axis=1, keepdims=True)
    qn = q * (1.0 / jnp.maximum(jnp.sqrt(qs), NORM_EPS))
    qb8 = (qn * SQ).astype(jnp.float8_e4m3fn)
    qk = qn * kb8_ref[...].astype(jnp.float32)
    dp = qk[:, :LN] + qk[:, LN:]
    dp_bf = dp.astype(jnp.bfloat16)

    rs = None
    cols = []
    for c in range(NJ):
        kb = kn8_ref[c * BN:(c + 1) * BN, :]
        s = jax.lax.dot_general(
            qb8, kb,
            (((1,), (1,)), ((), ())),
            preferred_element_type=jnp.float32,
        )
        e = jnp.exp2(s)
        acc = e[:, 0:LN]
        for cc in range(1, BN // LN):
            acc = acc + e[:, cc * LN:(cc + 1) * LN]
        rs = acc if rs is None else rs + acc
        cols.append(jnp.sum(e, axis=0)[None, :])

    colp_ref[pl.ds(i, 1)] = jnp.concatenate(cols, axis=1)[None]

    ones = jnp.ones((LN, LN), jnp.bfloat16)
    den = jax.lax.dot_general(
        rs.astype(jnp.bfloat16), ones,
        (((1,), (0,)), ((), ())),
        preferred_element_type=jnp.float32,
    )
    drep = jax.lax.dot_general(
        dp_bf, ones,
        (((1,), (0,)), ((), ())),
        preferred_element_type=jnp.float32,
    )
    dexp = jnp.exp2(drep * SQ)
    lq = -jnp.log(dexp / den + EPS)
    rl_ref[pl.ds(i, 1)] = jnp.sum(lq, axis=0)[None, None, :]

    ones_row = jnp.ones((1, LN), jnp.bfloat16)
    dlane_ref[pl.ds(i, 1)] = jax.lax.dot_general(
        ones_row, dp_bf,
        (((1,), (1,)), ((), ())),
        preferred_element_type=jnp.float32,
    )[None]

    @pl.when(i == NI - 1)
    def _():
        den_kq = colp_ref[0, 0, :][None, :]
        for r in range(1, NI):
            den_kq = den_kq + colp_ref[r, 0, :][None, :]
        lk_sum = jnp.float32(0.0)
        for r in range(NI):
            dex = jnp.exp2(dlane_ref[r, 0, :][None, :] * SQ)
            seg = den_kq[:, r * BM:(r + 1) * BM]
            lk_sum = lk_sum + jnp.sum(-jnp.log(dex / seg + EPS))
        rl_sum = jnp.float32(0.0)
        for r in range(NI):
            rl_sum = rl_sum + jnp.sum(rl_ref[r, 0, :])
        o_ref[...] = jnp.reshape(
            (rl_sum * (1.0 / LN) + lk_sum) * (1.0 / B), (1, 1))


def kernel(q, k):
    kn8 = pl.pallas_call(
        _kprep_kernel,
        grid=(4,),
        in_specs=[pl.BlockSpec((B // 4, D), lambda i: (i, 0))],
        out_specs=pl.BlockSpec((B // 4, D), lambda i: (i, 0)),
        out_shape=jax.ShapeDtypeStruct((B, D), jnp.float8_e4m3fn),
        compiler_params=pltpu.CompilerParams(
            dimension_semantics=("arbitrary",),
        ),
    )(k)

    loss = pl.pallas_call(
        _main_kernel,
        grid=(NI,),
        in_specs=[
            pl.BlockSpec((BM, D), lambda i: (i, 0)),
            pl.BlockSpec((B, D), lambda i: (0, 0)),
            pl.BlockSpec((BM, D), lambda i: (i, 0)),
        ],
        out_specs=pl.BlockSpec((1, 1), lambda i: (0, 0)),
        out_shape=jax.ShapeDtypeStruct((1, 1), jnp.float32),
        scratch_shapes=[
            pltpu.VMEM((NI, 1, B), jnp.float32),
            pltpu.VMEM((NI, 1, LN), jnp.float32),
            pltpu.VMEM((NI, 1, BM), jnp.float32),
        ],
        compiler_params=pltpu.CompilerParams(
            dimension_semantics=("arbitrary",),
            vmem_limit_bytes=40 * 1024 * 1024,
        ),
    )(q, kn8, kn8)

    return jnp.reshape(loss, ())


# speedup vs baseline: 1.5374x; 1.0031x over previous
import jax
import jax.numpy as jnp
from jax.experimental import pallas as pl
from jax.experimental.pallas import tpu as pltpu

B = 8192
D = 256
TEMP = 0.05
EPS = 1e-5
NORM_EPS = 1e-12
LOG2E = 1.4426950408889634
SC = LOG2E / TEMP
SQ = SC ** 0.5

BM = 1024
BN = 2048
NI = B // BM
NJ = B // BN
LN = 128


def _kprep_kernel(k_ref, kn8_ref):
    k = k_ref[...]
    ks = jnp.sum(k * k, axis=1, keepdims=True)
    kn = k * (1.0 / jnp.maximum(jnp.sqrt(ks), NORM_EPS))
    kn8_ref[...] = (kn * SQ).astype(jnp.float8_e4m3fn)


def _main_kernel(q_ref, kn8_ref, kb8_ref, o_ref, colp_ref, rl_ref, dlane_ref):
    i = pl.program_id(0)
    q = q_ref[...]
    qs = jnp.sum(q * q, axis=1, keepdims=True)
    qn = q * (1.0 / jnp.maximum(jnp.sqrt(qs), NORM_EPS))
    qb8 = (qn * SQ).astype(jnp.float8_e4m3fn)
    qk = qn * kb8_ref[...].astype(jnp.float32)
    dp = qk[:, :LN] + qk[:, LN:]
    dp_bf = dp.astype(jnp.bfloat16)

    rs = None
    cols = []
    for c in range(NJ):
        kb = kn8_ref[c * BN:(c + 1) * BN, :]
        s = jax.lax.dot_general(
            qb8, kb,
            (((1,), (1,)), ((), ())),
            preferred_element_type=jnp.float32,
        )
        e = jnp.exp2(s)
        acc = e[:, 0:LN]
        for cc in range(1, BN // LN):
            acc = acc + e[:, cc * LN:(cc + 1) * LN]
        rs = acc if rs is None else rs + acc
        cols.append(jnp.sum(e, axis=0)[None, :])

    colp_ref[pl.ds(i, 1)] = jnp.concatenate(cols, axis=1)[None]

    ones = jnp.ones((LN, LN), jnp.bfloat16)
    den = jax.lax.dot_general(
        rs.astype(jnp.bfloat16), ones,
        (((1,), (0,)), ((), ())),
        preferred_element_type=jnp.float32,
    )
    drep = jax.lax.dot_general(
        dp_bf, ones,
        (((1,), (0,)), ((), ())),
        preferred_element_type=jnp.float32,
    )
    dexp = jnp.exp2(drep * SQ)
    lq = -jnp.log(dexp / den + EPS)
    rl_ref[pl.ds(i, 1)] = jnp.sum(lq, axis=0)[None, None, :]

    ones_row = jnp.ones((1, LN), jnp.bfloat16)
    dlane_ref[pl.ds(i, 1)] = jax.lax.dot_general(
        ones_row, dp_bf,
        (((1,), (1,)), ((), ())),
        preferred_element_type=jnp.float32,
    )[None]

    @pl.when(i == NI - 1)
    def _():
        den_kq = colp_ref[0, 0, :][None, :]
        for r in range(1, NI):
            den_kq = den_kq + colp_ref[r, 0, :][None, :]
        lk_sum = jnp.float32(0.0)
        for r in range(NI):
            dex = jnp.exp2(dlane_ref[r, 0, :][None, :] * SQ)
            seg = den_kq[:, r * BM:(r + 1) * BM]
            lk_sum = lk_sum + jnp.sum(-jnp.log(dex / seg + EPS))
        rl_sum = jnp.float32(0.0)
        for r in range(NI):
            rl_sum = rl_sum + jnp.sum(rl_ref[r, 0, :])
        o_ref[...] = jnp.reshape(
            (rl_sum * (1.0 / LN) + lk_sum) * (1.0 / B), (1, 1))


def kernel(q, k):
    kn8 = pl.pallas_call(
        _kprep_kernel,
        grid=(4,),
        in_specs=[pl.BlockSpec((B // 4, D), lambda i: (i, 0))],
        out_specs=pl.BlockSpec((B // 4, D), lambda i: (i, 0)),
        out_shape=jax.ShapeDtypeStruct((B, D), jnp.float8_e4m3fn),
        compiler_params=pltpu.CompilerParams(
            dimension_semantics=("arbitrary",),
        ),
    )(k)

    loss = pl.pallas_call(
        _main_kernel,
        grid=(NI,),
        in_specs=[
            pl.BlockSpec((BM, D), lambda i: (i, 0)),
            pl.BlockSpec((B, D), lambda i: (0, 0)),
            pl.BlockSpec((BM, D), lambda i: (i, 0)),
        ],
        out_specs=pl.BlockSpec((1, 1), lambda i: (0, 0)),
        out_shape=jax.ShapeDtypeStruct((1, 1), jnp.float32),
        scratch_shapes=[
            pltpu.VMEM((NI, 1, B), jnp.float32),
            pltpu.VMEM((NI, 1, LN), jnp.float32),
            pltpu.VMEM((NI, 1, BM), jnp.float32),
        ],
        compiler_params=pltpu.CompilerParams(
            dimension_semantics=("arbitrary",),
            vmem_limit_bytes=40 * 1024 * 1024,
        ),
    )(q, kn8, kn8)

    return jnp.reshape(loss, ())


# speedup vs baseline: 1.5722x; 1.0226x over previous
import jax
import jax.numpy as jnp
from jax.experimental import pallas as pl
from jax.experimental.pallas import tpu as pltpu

B = 8192
D = 256
TEMP = 0.05
EPS = 1e-5
NORM_EPS = 1e-12
LOG2E = 1.4426950408889634
SC = LOG2E / TEMP
SQ = SC ** 0.5

BM = 2048
BN = 2048
NI = B // BM
NJ = B // BN
LN = 128


def _kprep_kernel(k_ref, kn8_ref):
    k = k_ref[...]
    ks = jnp.sum(k * k, axis=1, keepdims=True)
    kn = k * (1.0 / jnp.maximum(jnp.sqrt(ks), NORM_EPS))
    kn8_ref[...] = (kn * SQ).astype(jnp.float8_e4m3fn)


def _main_kernel(q_ref, kn8_ref, kb8_ref, o_ref, colp_ref, rl_ref, dlane_ref):
    i = pl.program_id(0)
    q = q_ref[...]
    qs = jnp.sum(q * q, axis=1, keepdims=True)
    qn = q * (1.0 / jnp.maximum(jnp.sqrt(qs), NORM_EPS))
    qb8 = (qn * SQ).astype(jnp.float8_e4m3fn)
    qk = qn * kb8_ref[...].astype(jnp.float32)
    dp = qk[:, :LN] + qk[:, LN:]
    dp_bf = dp.astype(jnp.bfloat16)

    rs = None
    cols = []
    for c in range(NJ):
        kb = kn8_ref[c * BN:(c + 1) * BN, :]
        s = jax.lax.dot_general(
            qb8, kb,
            (((1,), (1,)), ((), ())),
            preferred_element_type=jnp.float32,
        )
        e = jnp.exp2(s)
        acc = e[:, 0:LN]
        for cc in range(1, BN // LN):
            acc = acc + e[:, cc * LN:(cc + 1) * LN]
        rs = acc if rs is None else rs + acc
        cols.append(jnp.sum(e, axis=0)[None, :])

    colp_ref[pl.ds(i, 1)] = jnp.concatenate(cols, axis=1)[None]

    ones = jnp.ones((LN, LN), jnp.bfloat16)
    den = jax.lax.dot_general(
        rs.astype(jnp.bfloat16), ones,
        (((1,), (0,)), ((), ())),
        preferred_element_type=jnp.float32,
    )
    drep = jax.lax.dot_general(
        dp_bf, ones,
        (((1,), (0,)), ((), ())),
        preferred_element_type=jnp.float32,
    )
    dexp = jnp.exp2(drep * SQ)
    lq = -jnp.log(dexp / den + EPS)
    rl_ref[pl.ds(i, 1)] = jnp.sum(lq, axis=0)[None, None, :]

    ones_row = jnp.ones((1, LN), jnp.bfloat16)
    dlane_ref[pl.ds(i, 1)] = jax.lax.dot_general(
        ones_row, dp_bf,
        (((1,), (1,)), ((), ())),
        preferred_element_type=jnp.float32,
    )[None]

    @pl.when(i == NI - 1)
    def _():
        den_kq = colp_ref[0, 0, :][None, :]
        for r in range(1, NI):
            den_kq = den_kq + colp_ref[r, 0, :][None, :]
        lk_sum = jnp.float32(0.0)
        for r in range(NI):
            dex = jnp.exp2(dlane_ref[r, 0, :][None, :] * SQ)
            seg = den_kq[:, r * BM:(r + 1) * BM]
            lk_sum = lk_sum + jnp.sum(-jnp.log(dex / seg + EPS))
        rl_sum = jnp.float32(0.0)
        for r in range(NI):
            rl_sum = rl_sum + jnp.sum(rl_ref[r, 0, :])
        o_ref[...] = jnp.reshape(
            (rl_sum * (1.0 / LN) + lk_sum) * (1.0 / B), (1, 1))


def kernel(q, k):
    kn8 = pl.pallas_call(
        _kprep_kernel,
        grid=(4,),
        in_specs=[pl.BlockSpec((B // 4, D), lambda i: (i, 0))],
        out_specs=pl.BlockSpec((B // 4, D), lambda i: (i, 0)),
        out_shape=jax.ShapeDtypeStruct((B, D), jnp.float8_e4m3fn),
        compiler_params=pltpu.CompilerParams(
            dimension_semantics=("arbitrary",),
        ),
    )(k)

    loss = pl.pallas_call(
        _main_kernel,
        grid=(NI,),
        in_specs=[
            pl.BlockSpec((BM, D), lambda i: (i, 0)),
            pl.BlockSpec((B, D), lambda i: (0, 0)),
            pl.BlockSpec((BM, D), lambda i: (i, 0)),
        ],
        out_specs=pl.BlockSpec((1, 1), lambda i: (0, 0)),
        out_shape=jax.ShapeDtypeStruct((1, 1), jnp.float32),
        scratch_shapes=[
            pltpu.VMEM((NI, 1, B), jnp.float32),
            pltpu.VMEM((NI, 1, LN), jnp.float32),
            pltpu.VMEM((NI, 1, BM), jnp.float32),
        ],
        compiler_params=pltpu.CompilerParams(
            dimension_semantics=("arbitrary",),
            vmem_limit_bytes=40 * 1024 * 1024,
        ),
    )(q, kn8, kn8)

    return jnp.reshape(loss, ())
